# baseline (speedup 1.0000x reference)
import functools

import jax
import jax.numpy as jnp
from jax.experimental import pallas as pl
from jax.experimental.pallas import tpu as pltpu
from jax.experimental.pallas import tpu_sc as plsc

_N = 256
_BB = 8
_NC = 2
_NS = 16
_BPW = _N // (_NC * _NS)


def _emb_body(rand_ref, col_ref, row_ref):
    r = rand_ref[...]
    n = r.shape[1]
    less = r[:, :, None] < r[:, None, :]
    kk = jax.lax.broadcasted_iota(jnp.int32, (1, n, n), 1)
    jj = jax.lax.broadcasted_iota(jnp.int32, (1, n, n), 2)
    tie = (r[:, :, None] == r[:, None, :]) & (kk < jj)
    rank = jnp.sum((less | tie).astype(jnp.int32), axis=1)
    ii = jax.lax.broadcasted_iota(jnp.int32, (1, n, n), 1)
    col_ref[...] = (rank[:, None, :] == ii).astype(col_ref.dtype)
    row_ref[...] = jnp.zeros_like(row_ref)


def _sc_copy_body(cost_hbm, out_hbm, b0, b1, si0, si1, so0, so1):
    cid = jax.lax.axis_index("c")
    sid = jax.lax.axis_index("s")
    base = (sid * _NC + cid) * _BPW
    bufs = (b0, b1)
    sin = (si0, si1)
    sout = (so0, so1)
    inh = {}
    outh = {}
    for k in range(2 * _BPW):
        p = k & 1
        bb = base + (k // 2)
        half = (k % 2) * (_N // 2)
        if k >= 2:
            outh[k - 2].wait()
        inh[k] = pltpu.async_copy(
            cost_hbm.at[bb, pl.ds(half, _N // 2)], bufs[p], sin[p])
        inh[k].wait()
        outh[k] = pltpu.async_copy(
            bufs[p], out_hbm.at[bb, pl.ds(half, _N // 2)], sout[p])
    outh[2 * _BPW - 2].wait()
    outh[2 * _BPW - 1].wait()


_sc_copy = functools.partial(
    pl.kernel,
    out_type=jax.ShapeDtypeStruct((_N, _N, _N), jnp.float32),
    mesh=plsc.VectorSubcoreMesh(core_axis_name="c", subcore_axis_name="s"),
    scratch_types=[
        pltpu.VMEM((_N // 2, _N), jnp.float32),
        pltpu.VMEM((_N // 2, _N), jnp.float32),
        pltpu.SemaphoreType.DMA,
        pltpu.SemaphoreType.DMA,
        pltpu.SemaphoreType.DMA,
        pltpu.SemaphoreType.DMA,
    ],
    compiler_params=pltpu.CompilerParams(
        needs_layout_passes=False, use_tc_tiling_on_sc=True),
)(_sc_copy_body)


def kernel(cost_matrix):
    b, n, _ = cost_matrix.shape
    rkey = jax.random.fold_in(jax.random.key(0), 1)
    rand = jax.random.uniform(rkey, (b, n), dtype=jnp.float32)
    cost_out = _sc_copy(cost_matrix)
    col_emb, row_emb = pl.pallas_call(
        _emb_body,
        grid=(b // _BB,),
        in_specs=[pl.BlockSpec((_BB, n), lambda i: (i, 0))],
        out_specs=[
            pl.BlockSpec((_BB, n, n), lambda i: (i, 0, 0)),
            pl.BlockSpec((_BB, n, n), lambda i: (i, 0, 0)),
        ],
        out_shape=[
            jax.ShapeDtypeStruct((b, n, n), cost_matrix.dtype),
            jax.ShapeDtypeStruct((b, n, n), cost_matrix.dtype),
        ],
    )(rand)
    return (row_emb, col_emb, cost_out)


# speedup vs baseline: 1.2504x; 1.2504x over previous
import functools

import jax
import jax.numpy as jnp
from jax.experimental import pallas as pl
from jax.experimental.pallas import tpu as pltpu
from jax.experimental.pallas import tpu_sc as plsc

_N = 256
_BB = 8
_NC = 2
_NS = 16
_BPW = _N // (_NC * _NS)


def _emb_body(rand_ref, col_ref, row_ref):
    r = rand_ref[...]
    n = r.shape[1]
    less = r[:, :, None] < r[:, None, :]
    kk = jax.lax.broadcasted_iota(jnp.int32, (1, n, n), 1)
    jj = jax.lax.broadcasted_iota(jnp.int32, (1, n, n), 2)
    tie = (r[:, :, None] == r[:, None, :]) & (kk < jj)
    rank = jnp.sum((less | tie).astype(jnp.int32), axis=1)
    ii = jax.lax.broadcasted_iota(jnp.int32, (1, n, n), 1)
    col_ref[...] = (rank[:, None, :] == ii).astype(col_ref.dtype)
    row_ref[...] = jnp.zeros_like(row_ref)


def _sc_copy_body(cost_hbm, out_hbm, b0, b1, si0, si1, so0, so1):
    cid = jax.lax.axis_index("c")
    sid = jax.lax.axis_index("s")
    base = (sid * _NC + cid) * _BPW
    bufs = (b0, b1)
    sin = (si0, si1)
    sout = (so0, so1)
    inh = {}
    outh = {}
    for k in range(2 * _BPW):
        p = k & 1
        bb = base + (k // 2)
        half = (k % 2) * (_N // 2)
        if k >= 2:
            outh[k - 2].wait()
        inh[k] = pltpu.async_copy(
            cost_hbm.at[bb, pl.ds(half, _N // 2)], bufs[p], sin[p])
        inh[k].wait()
        outh[k] = pltpu.async_copy(
            bufs[p], out_hbm.at[bb, pl.ds(half, _N // 2)], sout[p])
    outh[2 * _BPW - 2].wait()
    outh[2 * _BPW - 1].wait()


_sc_copy = functools.partial(
    pl.kernel,
    out_type=jax.ShapeDtypeStruct((_N, _N, _N), jnp.float32),
    mesh=plsc.VectorSubcoreMesh(core_axis_name="c", subcore_axis_name="s"),
    scratch_types=[
        pltpu.VMEM((_N // 2, _N), jnp.float32),
        pltpu.VMEM((_N // 2, _N), jnp.float32),
        pltpu.SemaphoreType.DMA,
        pltpu.SemaphoreType.DMA,
        pltpu.SemaphoreType.DMA,
        pltpu.SemaphoreType.DMA,
    ],
    compiler_params=pltpu.CompilerParams(
        needs_layout_passes=False, use_tc_tiling_on_sc=True),
)(_sc_copy_body)


def kernel(cost_matrix):
    b, n, _ = cost_matrix.shape
    rkey = jax.random.fold_in(jax.random.key(0), 1)
    rand = jax.random.uniform(rkey, (b, n), dtype=jnp.float32)
    cost_out = cost_matrix
    col_emb, row_emb = pl.pallas_call(
        _emb_body,
        grid=(b // _BB,),
        in_specs=[pl.BlockSpec((_BB, n), lambda i: (i, 0))],
        out_specs=[
            pl.BlockSpec((_BB, n, n), lambda i: (i, 0, 0)),
            pl.BlockSpec((_BB, n, n), lambda i: (i, 0, 0)),
        ],
        out_shape=[
            jax.ShapeDtypeStruct((b, n, n), cost_matrix.dtype),
            jax.ShapeDtypeStruct((b, n, n), cost_matrix.dtype),
        ],
    )(rand)
    return (row_emb, col_emb, cost_out)


# speedup vs baseline: 1.2572x; 1.0054x over previous
import jax
import jax.numpy as jnp
from jax.experimental import pallas as pl

_BB = 8


def _emb_body(rand_ref, col_ref, row_ref):
    r = rand_ref[...]
    n = r.shape[1]
    less = r[:, :, None] < r[:, None, :]
    kk = jax.lax.broadcasted_iota(jnp.int32, (1, n, n), 1)
    jj = jax.lax.broadcasted_iota(jnp.int32, (1, n, n), 2)
    tie = (r[:, :, None] == r[:, None, :]) & (kk < jj)
    rank = jnp.sum((less | tie).astype(jnp.int32), axis=1)
    ii = jax.lax.broadcasted_iota(jnp.int32, (1, n, n), 1)
    col_ref[...] = (rank[:, None, :] == ii).astype(col_ref.dtype)
    row_ref[...] = jnp.zeros_like(row_ref)


def kernel(cost_matrix):
    b, n, _ = cost_matrix.shape
    rkey = jax.random.fold_in(jax.random.key(0), 1)
    rand = jax.random.uniform(rkey, (b, n), dtype=jnp.float32)
    col_emb, row_emb = pl.pallas_call(
        _emb_body,
        grid=(b // _BB,),
        in_specs=[pl.BlockSpec((_BB, n), lambda i: (i, 0))],
        out_specs=[
            pl.BlockSpec((_BB, n, n), lambda i: (i, 0, 0)),
            pl.BlockSpec((_BB, n, n), lambda i: (i, 0, 0)),
        ],
        out_shape=[
            jax.ShapeDtypeStruct((b, n, n), cost_matrix.dtype),
            jax.ShapeDtypeStruct((b, n, n), cost_matrix.dtype),
        ],
    )(rand)
    return (row_emb, col_emb, cost_matrix)
